# baseline (speedup 1.0000x reference)
import functools

import jax
import jax.numpy as jnp
from jax import lax
from jax.experimental import pallas as pl
from jax.experimental.pallas import tpu as pltpu
from jax.experimental.pallas import tpu_sc as plsc

_RELATION_NUM = 40
_DIM = 128
_B = 16384
_REG = 1e-05

_NSUB = 16
_NW = 32
_PW = _B // _NW
_NV = _PW // 16
_HALF = _B // 2
_CH = 128


def _make_sort_gather(n):
    mesh = plsc.VectorSubcoreMesh(core_axis_name="c", subcore_axis_name="s")
    pw = n // _NW
    nv = pw // 16
    half = n // 2
    nch = pw // _CH

    @functools.partial(
        pl.kernel,
        mesh=mesh,
        out_type=(
            jax.ShapeDtypeStruct((n, _DIM), jnp.float32),
            jax.ShapeDtypeStruct((n, _DIM), jnp.float32),
            jax.ShapeDtypeStruct((n, _DIM), jnp.float32),
            jax.ShapeDtypeStruct((n,), jnp.int32),
        ),
        scratch_types=[
            pltpu.VMEM((pw,), jnp.int32),
            pltpu.VMEM((pw,), jnp.int32),
            pltpu.VMEM((pw,), jnp.int32),
            pltpu.VMEM((pw,), jnp.int32),
            pltpu.VMEM((pw + 16,), jnp.int32),
            pltpu.VMEM((pw,), jnp.int32),
            pltpu.VMEM((16,), jnp.int32),
            pltpu.VMEM((16,), jnp.int32),
            pltpu.VMEM((48,), jnp.int32),
            pltpu.VMEM((48,), jnp.int32),
            pltpu.VMEM((48,), jnp.int32),
            pltpu.VMEM((nch, _CH), jnp.int32),
            pltpu.VMEM((pw,), jnp.int32),
            pltpu.VMEM((pw,), jnp.int32),
            pltpu.VMEM((pw,), jnp.int32),
            pltpu.VMEM((pw,), jnp.int32),
            pltpu.VMEM((_NSUB * 48,), jnp.int32),
            [pltpu.VMEM((_CH, _DIM), jnp.float32) for _ in range(6)],
            pltpu.VMEM_SHARED((_NSUB * 48,), jnp.int32),
            [pltpu.SemaphoreType.DMA for _ in range(6)],
            pltpu.SemaphoreType.DMA,
        ],
        compiler_params=pltpu.CompilerParams(needs_layout_passes=False),
    )
    def sg(table, heads, ptails, ntails, rels,
           out_h, out_p, out_n, out_r,
           relv, headv, ptv, ntv, ordbuf, rankbuf, tmp16, chg16, cnt48,
           lst48, delta48, pos2d,
           hsort, psort, nsort, rsort, hall, bufs, hists_sh, sems, lsem):
        c = lax.axis_index("c")
        s = lax.axis_index("s")
        gbase = c * half + s * pw
        ld_r = pltpu.async_copy(rels.at[pl.ds(gbase, pw)], relv, lsem)
        ld_h = pltpu.async_copy(heads.at[pl.ds(gbase, pw)], headv, sems[0])
        ld_p = pltpu.async_copy(ptails.at[pl.ds(gbase, pw)], ptv, sems[1])
        ld_n = pltpu.async_copy(ntails.at[pl.ds(gbase, pw)], ntv, sems[2])
        ld_r.wait()

        lane = lax.broadcasted_iota(jnp.int32, (16,), 0)
        zero = jnp.zeros((16,), jnp.int32)

        def bc(x):
            return jnp.broadcast_to(x, (16,))

        cnt48[pl.ds(0, 16)] = zero
        cnt48[pl.ds(16, 16)] = zero
        cnt48[pl.ds(32, 16)] = zero

        def pass1(j, carry):
            v = relv[pl.ds(j * 16, 16)]
            ks, vs = plsc.sort_key_val(v, lane)
            tmp16[...] = ks
            prev = plsc.load_gather(tmp16, [jnp.maximum(lane - 1, 0)])
            chg = ((ks != prev) | (lane == 0)).astype(jnp.int32)
            chg16[...] = chg
            nxt = plsc.load_gather(chg16, [jnp.minimum(lane + 1, 15)])
            lastchg = plsc.cummax(lane * chg)
            runrank = lane - lastchg
            base = plsc.load_gather(cnt48, [ks])
            localrank = base + runrank
            endm = (nxt == 1) | (lane == 15)
            plsc.store_scatter(cnt48, [ks], localrank + 1, mask=endm)
            plsc.store_scatter(rankbuf, [bc(j * 16) + vs], localrank)
            return carry

        lax.fori_loop(0, nv, pass1, 0)

        cnt0 = cnt48[pl.ds(0, 16)]
        cnt1 = cnt48[pl.ds(16, 16)]
        cnt2 = cnt48[pl.ds(32, 16)]
        cs0 = bc(jnp.sum(cnt0))
        cs1 = bc(jnp.sum(cnt1))
        lst0 = plsc.cumsum(cnt0) - cnt0
        lst1 = plsc.cumsum(cnt1) - cnt1 + cs0
        lst2 = plsc.cumsum(cnt2) - cnt2 + cs0 + cs1
        lst48[pl.ds(0, 16)] = lst0
        lst48[pl.ds(16, 16)] = lst1
        lst48[pl.ds(32, 16)] = lst2

        def pass2(j, carry):
            v = relv[pl.ds(j * 16, 16)]
            rk = rankbuf[pl.ds(j * 16, 16)]
            basel = plsc.load_gather(lst48, [v])
            plsc.store_scatter(ordbuf, [basel + rk], lane + bc(j * 16))
            return carry

        lax.fori_loop(0, nv, pass2, 0)

        pltpu.sync_copy(cnt48, hists_sh.at[pl.ds(s * 48, 48)])
        plsc.subcore_barrier()
        pltpu.sync_copy(hists_sh, hall)

        t0 = t1 = t2 = p0 = p1 = p2 = zero
        for w in range(_NSUB):
            h0 = hall[pl.ds(w * 48, 16)]
            h1 = hall[pl.ds(w * 48 + 16, 16)]
            h2 = hall[pl.ds(w * 48 + 32, 16)]
            use = bc((w < s).astype(jnp.int32))
            t0 += h0
            t1 += h1
            t2 += h2
            p0 += h0 * use
            p1 += h1 * use
            p2 += h2 * use
        s0 = bc(jnp.sum(t0))
        s1 = bc(jnp.sum(t1))
        e0 = plsc.cumsum(t0) - t0
        e1 = plsc.cumsum(t1) - t1 + s0
        e2 = plsc.cumsum(t2) - t2 + s0 + s1
        basev = bc(c * half)
        delta48[pl.ds(0, 16)] = e0 + p0 + basev - lst0
        delta48[pl.ds(16, 16)] = e1 + p1 + basev - lst1
        delta48[pl.ds(32, 16)] = e2 + p2 + basev - lst2

        ld_h.wait()
        ld_p.wait()
        ld_n.wait()
        for j in range(nv):
            ordv = ordbuf[pl.ds(j * 16, 16)]
            rsv = plsc.load_gather(relv, [ordv])
            dv = plsc.load_gather(delta48, [rsv])
            pos2d[j // 8, pl.ds((j % 8) * 16, 16)] = dv + lane + bc(j * 16)
            hsort[pl.ds(j * 16, 16)] = plsc.load_gather(headv, [ordv])
            psort[pl.ds(j * 16, 16)] = plsc.load_gather(ptv, [ordv])
            nsort[pl.ds(j * 16, 16)] = plsc.load_gather(ntv, [ordv])
            rsort[pl.ds(j * 16, 16)] = rsv

        rel_descs = [
            pltpu.async_copy(rsort.at[pl.ds(ch * _CH, _CH)],
                             out_r.at[pos2d.at[ch]], lsem)
            for ch in range(nch)
        ]

        srcs = [hsort, psort, nsort]
        outs = [out_h, out_p, out_n]
        njobs = 3 * nch

        def job(k):
            t, ch = k % 3, k // 3
            return srcs[t], outs[t], ch

        dg = {}
        dsc = {}
        for k in range(min(6, njobs)):
            src, _, ch = job(k)
            dg[k] = pltpu.async_copy(table.at[src.at[pl.ds(ch * _CH, _CH)]],
                                     bufs[k], sems[k])
        for k in range(njobs):
            slot = k % 6
            dg[k].wait()
            _, out, ch = job(k)
            dsc[k] = pltpu.async_copy(bufs[slot], out.at[pos2d.at[ch]],
                                      sems[slot])
            if k + 6 < njobs:
                dsc[k].wait()
                src, _, ch2 = job(k + 6)
                dg[k + 6] = pltpu.async_copy(
                    table.at[src.at[pl.ds(ch2 * _CH, _CH)]],
                    bufs[slot], sems[slot])
        for k in range(max(0, njobs - 6), njobs):
            dsc[k].wait()
        for dsc_rel in rel_descs:
            dsc_rel.wait()

    return sg


_BK = 1024
_NB = _B // _BK


def _tc_body(rel_ref, hb_ref, pb_ref, nb_ref, rtab_ref, trans_ref, out_ref,
             s_ref, acc_ref):
    i = pl.program_id(0)
    rel = rel_ref[0, 0, :]
    rel3 = jnp.concatenate([rel, rel, rel], axis=0)
    lo = jnp.min(rel)
    hi = jnp.max(rel)

    s_ref[0:_BK, :] = hb_ref[...]
    s_ref[_BK:2 * _BK, :] = pb_ref[...]
    s_ref[2 * _BK:3 * _BK, :] = nb_ref[...]
    acc_ref[...] = jnp.zeros((3 * _BK, _DIM), jnp.float32)

    def body(r, carry):
        m = (rel3 == r).astype(jnp.float32)[:, None]
        w_r = trans_ref[r, :, :]
        acc_ref[...] += jnp.dot(s_ref[...] * m, w_r,
                                preferred_element_type=jnp.float32)
        return carry

    lax.fori_loop(lo, hi + 1, body, 0)

    th = acc_ref[0:_BK, :]
    tp = acc_ref[_BK:2 * _BK, :]
    tn = acc_ref[2 * _BK:3 * _BK, :]

    oh = (rel[:, None] == lax.broadcasted_iota(jnp.int32, (1, _RELATION_NUM), 1)
          ).astype(jnp.float32)
    remb = jnp.dot(oh, rtab_ref[...], preferred_element_type=jnp.float32)

    pos = jnp.sum(jnp.square(th + remb - tp), axis=1)
    neg = jnp.sum(jnp.square(th + remb - tn), axis=1)
    d = neg - pos
    ls = jnp.minimum(d, 0.0) - jnp.log1p(jnp.exp(-jnp.abs(d)))

    rows = lax.broadcasted_iota(jnp.int32, (8, _DIM), 0)
    cols = lax.broadcasted_iota(jnp.int32, (8, _DIM), 1)
    partial = (jnp.sum(ls) * (rows == 0) + jnp.sum(th * th) * (rows == 1)
               + jnp.sum(remb * remb) * (rows == 2)
               + jnp.sum(tp * tp) * (rows == 3)
               + jnp.sum(tn * tn) * (rows == 4)).astype(jnp.float32)

    @pl.when(i == 0)
    def _():
        out_ref[...] = jnp.zeros((8, _DIM), jnp.float32)

    out_ref[...] += partial


def _tc_call(rows_h, rows_p, rows_n, rel3d, rtab, trans, nb):
    return pl.pallas_call(
        _tc_body,
        grid=(nb,),
        in_specs=[
            pl.BlockSpec((1, 1, _BK), lambda i: (i, 0, 0)),
            pl.BlockSpec((_BK, _DIM), lambda i: (i, 0)),
            pl.BlockSpec((_BK, _DIM), lambda i: (i, 0)),
            pl.BlockSpec((_BK, _DIM), lambda i: (i, 0)),
            pl.BlockSpec((_RELATION_NUM, _DIM), lambda i: (0, 0)),
            pl.BlockSpec((_RELATION_NUM, _DIM, _DIM), lambda i: (0, 0, 0)),
        ],
        out_specs=pl.BlockSpec((8, _DIM), lambda i: (0, 0)),
        out_shape=jax.ShapeDtypeStruct((8, _DIM), jnp.float32),
        scratch_shapes=[
            pltpu.VMEM((3 * _BK, _DIM), jnp.float32),
            pltpu.VMEM((3 * _BK, _DIM), jnp.float32),
        ],
    )(rel3d, rows_h, rows_p, rows_n, rtab, trans)


def kernel(user_entity_table, relation_table, trans_matrix, heads, relations,
           positive_tails, negative_tails):
    nh = _B // 2
    sg = _make_sort_gather(nh)
    heads = heads.astype(jnp.int32)
    ptails = positive_tails.astype(jnp.int32)
    ntails = negative_tails.astype(jnp.int32)
    rels = relations.astype(jnp.int32)
    parts = []
    for h in range(2):
        sl = slice(h * nh, (h + 1) * nh)
        rows_h, rows_p, rows_n, rel_s = sg(
            user_entity_table, heads[sl], ptails[sl], ntails[sl], rels[sl])
        rel3d = rel_s.reshape(nh // _BK, 1, _BK)
        parts.append(_tc_call(rows_h, rows_p, rows_n, rel3d, relation_table,
                              trans_matrix, nh // _BK))
    tot = parts[0] + parts[1]
    kg = -tot[0, 0] / _B
    l2 = (tot[1, 0] + tot[2, 0] + tot[3, 0] + tot[4, 0]) / (2.0 * _B)
    return kg + _REG * l2
